# baseline (speedup 1.0000x reference)
import functools

import jax
import jax.numpy as jnp
from jax import lax
from jax.experimental import pallas as pl
from jax.experimental.pallas import tpu as pltpu
from jax.experimental.pallas import tpu_sc as plsc

D_MODEL = 64
SCALE = 8.0
NUM_CORES = 2
NUM_SUBCORES = 16
NUM_WORKERS = NUM_CORES * NUM_SUBCORES
IDXW = 128
CHUNK_ROWS = 4


@functools.lru_cache(maxsize=None)
def _make_kernel(num_rows: int):
    rpw = num_rows // NUM_WORKERS
    chunks = rpw // CHUNK_ROWS

    mesh = plsc.VectorSubcoreMesh(core_axis_name="c", subcore_axis_name="s")

    @functools.partial(
        pl.kernel,
        mesh=mesh,
        out_type=jax.ShapeDtypeStruct((num_rows, IDXW, D_MODEL), jnp.float32),
        scratch_types=[
            pltpu.VMEM((rpw, IDXW), jnp.int32),
            pltpu.VMEM((CHUNK_ROWS, IDXW, D_MODEL), jnp.float32),
            pltpu.SemaphoreType.DMA,
        ],
        compiler_params=pltpu.CompilerParams(use_tc_tiling_on_sc=False),
    )
    def emb(x_hbm, table_hbm, out_hbm, idx_v, rows_v, gsem):
        wid = lax.axis_index("s") * NUM_CORES + lax.axis_index("c")
        base = wid * rpw
        pltpu.sync_copy(x_hbm.at[pl.ds(base, rpw)], idx_v)

        def chunk_body(c, _):
            r0 = c * CHUNK_ROWS
            copies = [
                pltpu.async_copy(
                    table_hbm.at[idx_v.at[r0 + j]], rows_v.at[j], gsem)
                for j in range(CHUNK_ROWS)
            ]
            for cp in copies:
                cp.wait()

            def scale_body(r, _):
                for j in range(CHUNK_ROWS):
                    for q in range(D_MODEL // 16):
                        sl = (j, r, pl.ds(q * 16, 16))
                        rows_v[sl] = rows_v[sl] * SCALE
                return 0

            lax.fori_loop(0, IDXW, scale_body, 0)

            pltpu.sync_copy(rows_v, out_hbm.at[pl.ds(base + r0, CHUNK_ROWS)])
            return 0

        lax.fori_loop(0, chunks, chunk_body, 0)

    return emb


@jax.jit
def kernel(x, table):
    bt, seq = x.shape
    flat = bt * seq
    idx = x.reshape(flat // IDXW, IDXW).astype(jnp.int32)
    out = _make_kernel(flat // IDXW)(idx, table)
    return out.reshape(bt, seq, D_MODEL)
